# baseline (speedup 1.0000x reference)
import functools

import jax
import jax.numpy as jnp
from jax import lax
from jax.experimental import pallas as pl
from jax.experimental.pallas import tpu as pltpu
from jax.experimental.pallas import tpu_sc as plsc

_KO = 32


def _sc_label_gather(seqs, e_idx):
    B, L = seqs.shape
    K = e_idx.shape[2]
    N = B * L
    NW = 32
    rows_pw = N // NW

    mesh = plsc.VectorSubcoreMesh(core_axis_name="c", subcore_axis_name="s")

    @functools.partial(
        pl.kernel,
        mesh=mesh,
        compiler_params=pltpu.CompilerParams(needs_layout_passes=False),
        out_type=jax.ShapeDtypeStruct((B, _KO, L), jnp.int32),
        scratch_types=[
            pltpu.VMEM((L,), jnp.int32),
            pltpu.VMEM((rows_pw, K), jnp.int32),
            pltpu.VMEM((_KO, rows_pw), jnp.int32),
        ],
    )
    def k(seqs_hbm, eidx_hbm, out_hbm, seqs_v, eidx_v, g_v):
        wid = lax.axis_index("s") * 2 + lax.axis_index("c")
        row0 = wid * rows_pw
        b = row0 // L
        i0 = row0 % L
        pltpu.sync_copy(seqs_hbm.at[b], seqs_v)
        pltpu.sync_copy(eidx_hbm.at[b, pl.ds(i0, rows_pw)], eidx_v)

        jA = lax.broadcasted_iota(jnp.int32, (16,), 0)
        jB = jA + (K - 16)

        def body(i, carry):
            iv = jA * 0 + i
            eA = eidx_v[i, pl.ds(0, 16)]
            eA = jnp.where(jA == 0, i0 + i, eA)
            plsc.store_scatter(g_v, [jA, iv], plsc.load_gather(seqs_v, [eA]))
            eB = eidx_v[i, pl.ds(K - 16, 16)]
            plsc.store_scatter(g_v, [jB, iv], plsc.load_gather(seqs_v, [eB]))
            return carry

        lax.fori_loop(0, rows_pw, body, 0)
        pltpu.sync_copy(g_v, out_hbm.at[b, :, pl.ds(i0, rows_pw)])

    return k(seqs, e_idx)


_S = 10


def _tc_body(*refs):
    es = refs[:_S]
    g_ref, self_ref, mask_ref, gam_ref, bet_ref, out_ref = refs[_S:]
    b = pl.program_id(0)
    K = es[0].shape[1]
    CH = es[0].shape[2]
    A = self_ref.shape[1]
    R = self_ref.shape[2]

    Gtb = g_ref[0]
    U = 2 * A
    c_sub = lax.broadcasted_iota(jnp.int32, (U, R), 0) % A
    parts = []
    for e_ref in es:
        for h in range(CH // U):
            acc = jnp.zeros((U, R), jnp.float32)
            for j in range(1, K):
                gj = Gtb[j : j + 1, :]
                acc = acc + jnp.where(
                    gj == c_sub, e_ref[0, j, h * U : (h + 1) * U], 0.0)
            parts.append(jnp.sum(acc.reshape(U // A, A, R), axis=1))
    pair = jnp.concatenate(parts, axis=0)

    aa = self_ref[0] + pair
    mu = jnp.mean(aa, axis=0, keepdims=True)
    var = jnp.mean((aa - mu) ** 2, axis=0, keepdims=True)
    gam = jnp.broadcast_to(gam_ref[...], (A, R))
    bet = jnp.broadcast_to(bet_ref[...], (A, R))
    aa = (aa - mu) * lax.rsqrt(var + 1e-5) * gam + bet

    neg = -aa
    mx = jnp.max(neg, axis=0, keepdims=True)
    lse = jnp.log(jnp.sum(jnp.exp(neg - mx), axis=0, keepdims=True))
    s_oh = Gtb[0:1, :] == lax.broadcasted_iota(jnp.int32, (A, R), 0)
    negs = jnp.sum(jnp.where(s_oh, neg, 0.0), axis=0, keepdims=True)
    mrow = mask_ref[pl.ds(b, 1), :]
    logp = (negs - mx - lse) * mrow
    num = jnp.sum(logp, axis=(0, 1), keepdims=True)
    den = jnp.sum(mrow, axis=(0, 1), keepdims=True)
    out_ref[0] = num / den


def kernel(self_etab, etab, E_idx, seqs, x_mask, ln_gamma, ln_beta):
    B, L, K, AA = etab.shape
    A = self_etab.shape[-1]
    R = L

    Gt = _sc_label_gather(seqs.astype(jnp.int32), E_idx)

    etab_t = jnp.transpose(etab, (0, 2, 3, 1))
    self_t = jnp.transpose(self_etab, (0, 2, 1))
    gam = ln_gamma.astype(jnp.float32)[:, None]
    bet = ln_beta.astype(jnp.float32)[:, None]

    CH = AA // _S
    etab_specs = [
        pl.BlockSpec((1, K, CH, R), functools.partial(lambda s, b: (b, 0, s, 0), s))
        for s in range(_S)
    ]
    nlpl = pl.pallas_call(
        _tc_body,
        grid=(B,),
        in_specs=etab_specs + [
            pl.BlockSpec((1, _KO, R), lambda b: (b, 0, 0)),
            pl.BlockSpec((1, A, R), lambda b: (b, 0, 0)),
            pl.BlockSpec((B, R), lambda b: (0, 0)),
            pl.BlockSpec((A, 1), lambda b: (0, 0)),
            pl.BlockSpec((A, 1), lambda b: (0, 0)),
        ],
        out_specs=pl.BlockSpec((1, 1, 1), lambda b: (b, 0, 0)),
        out_shape=jax.ShapeDtypeStruct((B, 1, 1), jnp.float32),
    )(*([etab_t] * _S), Gt, self_t, x_mask, gam, bet)

    return -jnp.mean(nlpl)


# speedup vs baseline: 1.0480x; 1.0480x over previous
import functools

import jax
import jax.numpy as jnp
from jax import lax
from jax.experimental import pallas as pl
from jax.experimental.pallas import tpu as pltpu
from jax.experimental.pallas import tpu_sc as plsc

_KO = 32


def _sc_label_gather(seqs, e_idx):
    B, L = seqs.shape
    K = e_idx.shape[2]
    N = B * L
    NW = 32
    rows_pw = N // NW

    mesh = plsc.VectorSubcoreMesh(core_axis_name="c", subcore_axis_name="s")

    @functools.partial(
        pl.kernel,
        mesh=mesh,
        compiler_params=pltpu.CompilerParams(needs_layout_passes=False),
        out_type=jax.ShapeDtypeStruct((B, _KO, L), jnp.int32),
        scratch_types=[
            pltpu.VMEM((L,), jnp.int32),
            pltpu.VMEM((rows_pw, K), jnp.int32),
            pltpu.VMEM((_KO, rows_pw), jnp.int32),
        ],
    )
    def k(seqs_hbm, eidx_hbm, out_hbm, seqs_v, eidx_v, g_v):
        wid = lax.axis_index("s") * 2 + lax.axis_index("c")
        row0 = wid * rows_pw
        b = row0 // L
        i0 = row0 % L
        pltpu.sync_copy(seqs_hbm.at[b], seqs_v)
        pltpu.sync_copy(eidx_hbm.at[b, pl.ds(i0, rows_pw)], eidx_v)

        jA = lax.broadcasted_iota(jnp.int32, (16,), 0)
        jB = jA + (K - 16)

        def body(i, carry):
            iv = jA * 0 + i
            eA = eidx_v[i, pl.ds(0, 16)]
            eA = jnp.where(jA == 0, i0 + i, eA)
            plsc.store_scatter(g_v, [jA, iv], plsc.load_gather(seqs_v, [eA]))
            eB = eidx_v[i, pl.ds(K - 16, 16)]
            plsc.store_scatter(g_v, [jB, iv], plsc.load_gather(seqs_v, [eB]))
            return carry

        lax.fori_loop(0, rows_pw, body, 0)
        pltpu.sync_copy(g_v, out_hbm.at[b, :, pl.ds(i0, rows_pw)])

    return k(seqs, e_idx)


_S = 15


def _tc_body(*refs):
    es = refs[:_S]
    g_ref, self_ref, mask_ref, gam_ref, bet_ref, out_ref = refs[_S:]
    b = pl.program_id(0)
    KC = es[0].shape[1]
    AA = es[0].shape[2]
    A = self_ref.shape[1]
    R = self_ref.shape[2]

    Gtb = g_ref[0]
    U = 2 * A
    c_sub = lax.broadcasted_iota(jnp.int32, (U, R), 0) % A
    parts = []
    for h in range(AA // U):
        acc = jnp.zeros((U, R), jnp.float32)
        for s, e_ref in enumerate(es):
            for jj in range(KC):
                j = s * KC + jj
                if j == 0:
                    continue
                gj = Gtb[j : j + 1, :]
                acc = acc + jnp.where(
                    gj == c_sub, e_ref[0, jj, h * U : (h + 1) * U], 0.0)
        parts.append(jnp.sum(acc.reshape(U // A, A, R), axis=1))
    pair = jnp.concatenate(parts, axis=0)

    aa = self_ref[0] + pair
    mu = jnp.mean(aa, axis=0, keepdims=True)
    var = jnp.mean((aa - mu) ** 2, axis=0, keepdims=True)
    gam = jnp.broadcast_to(gam_ref[...], (A, R))
    bet = jnp.broadcast_to(bet_ref[...], (A, R))
    aa = (aa - mu) * lax.rsqrt(var + 1e-5) * gam + bet

    neg = -aa
    mx = jnp.max(neg, axis=0, keepdims=True)
    lse = jnp.log(jnp.sum(jnp.exp(neg - mx), axis=0, keepdims=True))
    s_oh = Gtb[0:1, :] == lax.broadcasted_iota(jnp.int32, (A, R), 0)
    negs = jnp.sum(jnp.where(s_oh, neg, 0.0), axis=0, keepdims=True)
    mrow = mask_ref[pl.ds(b, 1), :]
    logp = (negs - mx - lse) * mrow
    num = jnp.sum(logp, axis=(0, 1), keepdims=True)
    den = jnp.sum(mrow, axis=(0, 1), keepdims=True)
    out_ref[0] = num / den


def kernel(self_etab, etab, E_idx, seqs, x_mask, ln_gamma, ln_beta):
    B, L, K, AA = etab.shape
    A = self_etab.shape[-1]
    R = L

    Gt = _sc_label_gather(seqs.astype(jnp.int32), E_idx)

    etab_t = jnp.transpose(etab, (0, 2, 3, 1))
    self_t = jnp.transpose(self_etab, (0, 2, 1))
    gam = ln_gamma.astype(jnp.float32)[:, None]
    bet = ln_beta.astype(jnp.float32)[:, None]

    KC = K // _S
    etab_specs = [
        pl.BlockSpec((1, KC, AA, R), functools.partial(lambda s, b: (b, s, 0, 0), s))
        for s in range(_S)
    ]
    nlpl = pl.pallas_call(
        _tc_body,
        grid=(B,),
        in_specs=etab_specs + [
            pl.BlockSpec((1, _KO, R), lambda b: (b, 0, 0)),
            pl.BlockSpec((1, A, R), lambda b: (b, 0, 0)),
            pl.BlockSpec((B, R), lambda b: (0, 0)),
            pl.BlockSpec((A, 1), lambda b: (0, 0)),
            pl.BlockSpec((A, 1), lambda b: (0, 0)),
        ],
        out_specs=pl.BlockSpec((1, 1, 1), lambda b: (b, 0, 0)),
        out_shape=jax.ShapeDtypeStruct((B, 1, 1), jnp.float32),
    )(*([etab_t] * _S), Gt, self_t, x_mask, gam, bet)

    return -jnp.mean(nlpl)


# speedup vs baseline: 1.0727x; 1.0236x over previous
import functools

import jax
import jax.numpy as jnp
from jax import lax
from jax.experimental import pallas as pl
from jax.experimental.pallas import tpu as pltpu
from jax.experimental.pallas import tpu_sc as plsc

_KO = 32


def _sc_label_gather(seqs, e_idx):
    B, L = seqs.shape
    K = e_idx.shape[2]
    N = B * L
    NW = 32
    rows_pw = N // NW

    mesh = plsc.VectorSubcoreMesh(core_axis_name="c", subcore_axis_name="s")

    @functools.partial(
        pl.kernel,
        mesh=mesh,
        compiler_params=pltpu.CompilerParams(needs_layout_passes=False),
        out_type=jax.ShapeDtypeStruct((B, _KO, L), jnp.int32),
        scratch_types=[
            pltpu.VMEM((L,), jnp.int32),
            pltpu.VMEM((rows_pw, K), jnp.int32),
            pltpu.VMEM((_KO, rows_pw), jnp.int32),
        ],
    )
    def k(seqs_hbm, eidx_hbm, out_hbm, seqs_v, eidx_v, g_v):
        wid = lax.axis_index("s") * 2 + lax.axis_index("c")
        row0 = wid * rows_pw
        b = row0 // L
        i0 = row0 % L
        pltpu.sync_copy(seqs_hbm.at[b], seqs_v)
        pltpu.sync_copy(eidx_hbm.at[b, pl.ds(i0, rows_pw)], eidx_v)

        jA = lax.broadcasted_iota(jnp.int32, (16,), 0)
        jB = jA + (K - 16)

        def body(i, carry):
            iv = jA * 0 + i
            eA = eidx_v[i, pl.ds(0, 16)]
            eA = jnp.where(jA == 0, i0 + i, eA)
            plsc.store_scatter(g_v, [jA, iv], plsc.load_gather(seqs_v, [eA]))
            eB = eidx_v[i, pl.ds(K - 16, 16)]
            plsc.store_scatter(g_v, [jB, iv], plsc.load_gather(seqs_v, [eB]))
            return carry

        lax.fori_loop(0, rows_pw, body, 0)
        pltpu.sync_copy(g_v, out_hbm.at[b, :, pl.ds(i0, rows_pw)])

    return k(seqs, e_idx)


_S = 15


def _tc_body(*refs):
    es = refs[:_S]
    g_ref, self_ref, mask_ref, gam_ref, bet_ref, out_ref = refs[_S:]
    b = pl.program_id(0)
    KC = es[0].shape[1]
    AA = es[0].shape[2]
    A = self_ref.shape[1]
    R = self_ref.shape[2]

    Gtb = g_ref[0]
    U = 2 * A
    c_sub = lax.broadcasted_iota(jnp.int32, (U, R), 0) % A
    parts = []
    for h in range(AA // U):
        acc = jnp.zeros((U, R), jnp.float32)
        for s, e_ref in enumerate(es):
            for jj in range(e_ref.shape[1]):
                j = 1 if s == 0 else 2 * s + jj
                gj = Gtb[j : j + 1, :]
                acc = acc + jnp.where(
                    gj == c_sub, e_ref[0, jj, h * U : (h + 1) * U], 0.0)
        parts.append(jnp.sum(acc.reshape(U // A, A, R), axis=1))
    pair = jnp.concatenate(parts, axis=0)

    aa = self_ref[0] + pair
    mu = jnp.mean(aa, axis=0, keepdims=True)
    var = jnp.mean((aa - mu) ** 2, axis=0, keepdims=True)
    gam = jnp.broadcast_to(gam_ref[...], (A, R))
    bet = jnp.broadcast_to(bet_ref[...], (A, R))
    aa = (aa - mu) * lax.rsqrt(var + 1e-5) * gam + bet

    neg = -aa
    mx = jnp.max(neg, axis=0, keepdims=True)
    lse = jnp.log(jnp.sum(jnp.exp(neg - mx), axis=0, keepdims=True))
    s_oh = Gtb[0:1, :] == lax.broadcasted_iota(jnp.int32, (A, R), 0)
    negs = jnp.sum(jnp.where(s_oh, neg, 0.0), axis=0, keepdims=True)
    mrow = mask_ref[pl.ds(b, 1), :]
    logp = (negs - mx - lse) * mrow
    num = jnp.sum(logp, axis=(0, 1), keepdims=True)
    den = jnp.sum(mrow, axis=(0, 1), keepdims=True)
    out_ref[0] = num / den


def kernel(self_etab, etab, E_idx, seqs, x_mask, ln_gamma, ln_beta):
    B, L, K, AA = etab.shape
    A = self_etab.shape[-1]
    R = L

    Gt = _sc_label_gather(seqs.astype(jnp.int32), E_idx)

    etab_t = jnp.transpose(etab, (0, 2, 3, 1))
    self_t = jnp.transpose(self_etab, (0, 2, 1))
    gam = ln_gamma.astype(jnp.float32)[:, None]
    bet = ln_beta.astype(jnp.float32)[:, None]

    etab_specs = [
        pl.BlockSpec((1, 1, AA, R), lambda b: (b, 1, 0, 0))
    ] + [
        pl.BlockSpec((1, 2, AA, R), functools.partial(lambda s, b: (b, s, 0, 0), s))
        for s in range(1, _S)
    ]
    nlpl = pl.pallas_call(
        _tc_body,
        grid=(B,),
        in_specs=etab_specs + [
            pl.BlockSpec((1, _KO, R), lambda b: (b, 0, 0)),
            pl.BlockSpec((1, A, R), lambda b: (b, 0, 0)),
            pl.BlockSpec((B, R), lambda b: (0, 0)),
            pl.BlockSpec((A, 1), lambda b: (0, 0)),
            pl.BlockSpec((A, 1), lambda b: (0, 0)),
        ],
        out_specs=pl.BlockSpec((1, 1, 1), lambda b: (b, 0, 0)),
        out_shape=jax.ShapeDtypeStruct((B, 1, 1), jnp.float32),
    )(*([etab_t] * _S), Gt, self_t, x_mask, gam, bet)

    return -jnp.mean(nlpl)
